# Initial kernel scaffold; baseline (speedup 1.0000x reference)
import jax
import jax.numpy as jnp
from jax.experimental import pallas as pl


def kernel(x, kohonen_weights, grossberg_weights):
    raise NotImplementedError("write your pallas kernel here")


# speedup vs baseline: 5.1945x; 5.1945x over previous
import jax
import jax.numpy as jnp
from jax.experimental import pallas as pl
from jax.experimental.pallas import tpu as pltpu
from jax.experimental.pallas import tpu_sc as plsc

_B = 16384
_D = 32
_K = 8192
_N = 10
_NP = 128
_BM = 256
_W = 128


def _winners_body(x_ref, kwt_ref, x2_ref, w2_ref, win_ref):
    xb = x_ref[...]
    kwt = kwt_ref[...]
    xq = xb.astype(jnp.bfloat16).astype(jnp.float32)
    wq = kwt.astype(jnp.bfloat16).astype(jnp.float32)
    cross = jax.lax.dot_general(
        xq, wq, (((1,), (0,)), ((), ())),
        precision=jax.lax.Precision.HIGHEST,
        preferred_element_type=jnp.float32)
    d2 = jnp.maximum(x2_ref[...] + w2_ref[...] - 2.0 * cross, 0.0)
    dist = jnp.where(d2 == 0.0, 0.0, d2 * jax.lax.rsqrt(d2))
    h = _K // 2
    d0 = dist[:, :h]
    d1 = dist[:, h:]
    m0 = jnp.min(d0, axis=1, keepdims=True)
    m1 = jnp.min(d1, axis=1, keepdims=True)
    iota = jax.lax.broadcasted_iota(jnp.int32, d0.shape, 1)
    i0 = jnp.min(jnp.where(d0 == m0, iota, _K), axis=1)
    i1 = jnp.min(jnp.where(d1 == m1, iota, _K), axis=1) + h
    q0 = m0.astype(jnp.bfloat16).astype(jnp.float32)
    idx = jnp.where(m1[:, 0] < q0[:, 0], i1, i0)
    win_ref[...] = idx


def _winners(x, kwt, x2, w2):
    return pl.pallas_call(
        _winners_body,
        grid=(_B // _BM,),
        in_specs=[
            pl.BlockSpec((_BM, _D), lambda i: (i, 0)),
            pl.BlockSpec((_D, _K), lambda i: (0, 0)),
            pl.BlockSpec((_BM, 1), lambda i: (i, 0)),
            pl.BlockSpec((1, _K), lambda i: (0, 0)),
        ],
        out_specs=pl.BlockSpec((_BM,), lambda i: (i,)),
        out_shape=jax.ShapeDtypeStruct((_B,), jnp.int32),
    )(x, kwt, x2, w2)


def _decode(table, indices):
    @pl.kernel(
        out_type=jax.ShapeDtypeStruct((_B, _NP), jnp.float32),
        mesh=plsc.VectorSubcoreMesh(
            core_axis_name="core", subcore_axis_name="subcore"),
    )
    def kern(tab_hbm, idx_hbm, o_hbm):
        def body(i_vmem, o_vmem):
            pltpu.sync_copy(tab_hbm.at[i_vmem.at[0]], o_vmem)

        pltpu.emit_pipeline(
            body,
            grid=(_B // _W,),
            in_specs=[pl.BlockSpec((1, _W), index_map=lambda i: (0, i))],
            out_specs=[pl.BlockSpec((_W, _NP), index_map=lambda i: (i, 0))],
            core_axis_name=("core", "subcore"),
            dimension_semantics=(pltpu.PARALLEL,),
        )(idx_hbm, o_hbm)

    return kern(table, indices)


def kernel(x, kohonen_weights, grossberg_weights):
    xf = x.reshape(x.shape[0], -1)
    x2 = jnp.sum(xf * xf, axis=1, keepdims=True)
    w2 = jnp.sum(kohonen_weights * kohonen_weights, axis=1)[None, :]
    winners = _winners(xf, kohonen_weights.T, x2, w2)
    table = jnp.pad(grossberg_weights.T, ((0, 0), (0, _NP - _N)))
    out = _decode(table, winners.reshape(1, _B))
    return (out[:, :_N], winners)


# speedup vs baseline: 8.6693x; 1.6689x over previous
import jax
import jax.numpy as jnp
from jax.experimental import pallas as pl
from jax.experimental.pallas import tpu as pltpu
from jax.experimental.pallas import tpu_sc as plsc

_B = 16384
_D = 32
_K = 8192
_N = 10
_NP = 128
_BM = 256
_W = 128


def _winners_body(x_ref, kwt_ref, x2_ref, w2_ref, win_ref):
    xb = x_ref[...]
    kwt = kwt_ref[...]
    xq = xb.astype(jnp.bfloat16)
    wq = kwt.astype(jnp.bfloat16)
    cross = jax.lax.dot_general(
        xq, wq, (((1,), (0,)), ((), ())),
        preferred_element_type=jnp.float32)
    d2 = jnp.maximum(x2_ref[...] + w2_ref[...] - 2.0 * cross, 0.0)
    dist = jnp.where(d2 == 0.0, 0.0, d2 * jax.lax.rsqrt(d2))
    h = _K // 2
    d0 = dist[:, :h]
    d1 = dist[:, h:]
    m0 = jnp.min(d0, axis=1, keepdims=True)
    m1 = jnp.min(d1, axis=1, keepdims=True)
    iota = jax.lax.broadcasted_iota(jnp.int32, d0.shape, 1)
    i0 = jnp.min(jnp.where(d0 == m0, iota, _K), axis=1)
    i1 = jnp.min(jnp.where(d1 == m1, iota, _K), axis=1) + h
    q0 = m0.astype(jnp.bfloat16).astype(jnp.float32)
    idx = jnp.where(m1[:, 0] < q0[:, 0], i1, i0)
    win_ref[...] = idx


def _winners(x, kwt, x2, w2):
    return pl.pallas_call(
        _winners_body,
        grid=(_B // _BM,),
        in_specs=[
            pl.BlockSpec((_BM, _D), lambda i: (i, 0)),
            pl.BlockSpec((_D, _K), lambda i: (0, 0)),
            pl.BlockSpec((_BM, 1), lambda i: (i, 0)),
            pl.BlockSpec((1, _K), lambda i: (0, 0)),
        ],
        out_specs=pl.BlockSpec((_BM,), lambda i: (i,)),
        out_shape=jax.ShapeDtypeStruct((_B,), jnp.int32),
    )(x, kwt, x2, w2)


def _decode(table, indices):
    @pl.kernel(
        out_type=jax.ShapeDtypeStruct((_B, _NP), jnp.float32),
        mesh=plsc.VectorSubcoreMesh(
            core_axis_name="core", subcore_axis_name="subcore"),
    )
    def kern(tab_hbm, idx_hbm, o_hbm):
        def body(i_vmem, o_vmem):
            pltpu.sync_copy(tab_hbm.at[i_vmem.at[0]], o_vmem)

        pltpu.emit_pipeline(
            body,
            grid=(_B // _W,),
            in_specs=[pl.BlockSpec((1, _W), index_map=lambda i: (0, i))],
            out_specs=[pl.BlockSpec((_W, _NP), index_map=lambda i: (i, 0))],
            core_axis_name=("core", "subcore"),
            dimension_semantics=(pltpu.PARALLEL,),
        )(idx_hbm, o_hbm)

    return kern(table, indices)


def kernel(x, kohonen_weights, grossberg_weights):
    xf = x.reshape(x.shape[0], -1)
    x2 = jnp.sum(xf * xf, axis=1, keepdims=True)
    w2 = jnp.sum(kohonen_weights * kohonen_weights, axis=1)[None, :]
    winners = _winners(xf, kohonen_weights.T, x2, w2)
    table = jnp.pad(grossberg_weights.T, ((0, 0), (0, _NP - _N)))
    out = _decode(table, winners.reshape(1, _B))
    return (out[:, :_N], winners)
